# baseline (speedup 1.0000x reference)
import functools

import jax
import jax.numpy as jnp
from jax import lax
from jax.experimental import pallas as pl
from jax.experimental.pallas import tpu as pltpu
from jax.experimental.pallas import tpu_sc as plsc

_NUM_CAMS = 8
_NUM_IDS = 1500
_D = 2048

_NW = 32
_G = 3
_NCHUNKS = _NUM_IDS // _G
_STEPS = (_NCHUNKS + _NW - 1) // _NW

_LANES = _D // 16


def _lanesum(acc):
    idx = lax.iota(jnp.int32, 16)
    for k in (1, 2, 4, 8):
        perm = acc.at[idx ^ k].get(mode="promise_in_bounds")
        acc = acc + perm
    return acc


def _rsqrt16(s):
    i = lax.bitcast_convert_type(s, jnp.int32)
    i = jnp.int32(0x5F3759DF) - lax.shift_right_arithmetic(i, 1)
    r = lax.bitcast_convert_type(i, jnp.float32)
    for _ in range(3):
        r = r * (1.5 - 0.5 * s * r * r)
    return r


def _sumsq_rows(buf):
    invs = []
    for g in range(_G):
        for cam in range(_NUM_CAMS):
            def sumsq(j, acc):
                v = buf[g, cam, pl.ds(j * 16, 16)]
                return acc + v * v

            acc = lax.fori_loop(0, _LANES, sumsq,
                                jnp.zeros((16,), jnp.float32), unroll=8)
            s = _lanesum(acc)
            invs.append(1.0 / (s * _rsqrt16(s) + 1e-12))
    return invs


def _scale_rows(buf, invs):
    for g in range(_G):
        for cam in range(_NUM_CAMS):
            inv = invs[g * _NUM_CAMS + cam]

            def scale(j, carry):
                v = buf[g, cam, pl.ds(j * 16, 16)]
                buf[g, cam, pl.ds(j * 16, 16)] = v * inv
                return carry

            lax.fori_loop(0, _LANES, scale, 0, unroll=8)


def _sc_body(x_hbm, cp_hbm, nm_hbm, buf0, buf1, in_s0, in_s1, cp_s0, cp_s1,
             nm_s0, nm_s1):
    wid = lax.axis_index("s") * 2 + lax.axis_index("c")
    bufs = (buf0, buf1)
    in_sems = (in_s0, in_s1)
    cp_sems = (cp_s0, cp_s1)
    nm_sems = (nm_s0, nm_s1)

    def src(cid):
        return x_hbm.at[pl.ds(cid * _G, _G)]

    for b in range(2):
        cid = b * _NW + wid

        @pl.when(cid < _NCHUNKS)
        def _(b=b, cid=cid):
            pltpu.async_copy(src(cid), bufs[b], in_sems[b])

    def pair(i, carry):
        for b in range(2):
            t = i * 2 + b
            cid = t * _NW + wid

            @pl.when(cid < _NCHUNKS)
            def _(b=b, cid=cid):
                buf = bufs[b]
                pltpu.async_copy(src(cid), cp_hbm.at[pl.ds(cid * _G, _G)],
                                 cp_sems[b])
                pltpu.make_async_copy(src(cid), buf, in_sems[b]).wait()
                invs = _sumsq_rows(buf)
                _scale_rows(buf, invs)
                pltpu.async_copy(buf, nm_hbm.at[pl.ds(cid * _G, _G)],
                                 nm_sems[b])

            cid2 = cid + 2 * _NW

            @pl.when(cid2 < _NCHUNKS)
            def _(b=b, cid=cid, cid2=cid2):
                pltpu.make_async_copy(src(cid), cp_hbm.at[pl.ds(cid * _G, _G)],
                                      cp_sems[b]).wait()
                pltpu.make_async_copy(bufs[b], nm_hbm.at[pl.ds(cid * _G, _G)],
                                      nm_sems[b]).wait()
                pltpu.async_copy(src(cid2), bufs[b], in_sems[b])

        return carry

    lax.fori_loop(0, _STEPS // 2, pair, 0)

    for t in range(max(0, _STEPS - 3), _STEPS):
        cid = t * _NW + wid

        @pl.when((cid < _NCHUNKS) & (cid + 2 * _NW >= _NCHUNKS))
        def _(t=t, cid=cid):
            b = t % 2
            pltpu.make_async_copy(src(cid), cp_hbm.at[pl.ds(cid * _G, _G)],
                                  cp_sems[b]).wait()
            pltpu.make_async_copy(bufs[b], nm_hbm.at[pl.ds(cid * _G, _G)],
                                  nm_sems[b]).wait()


def _sc_normalize(xt):
    mesh = plsc.VectorSubcoreMesh(core_axis_name="c", subcore_axis_name="s")
    shape = jax.ShapeDtypeStruct((_NUM_IDS, _NUM_CAMS, _D), jnp.float32)
    k = functools.partial(
        pl.kernel,
        mesh=mesh,
        out_type=[shape, shape],
        scratch_types=[
            pltpu.VMEM((_G, _NUM_CAMS, _D), jnp.float32),
            pltpu.VMEM((_G, _NUM_CAMS, _D), jnp.float32),
            pltpu.SemaphoreType.DMA,
            pltpu.SemaphoreType.DMA,
            pltpu.SemaphoreType.DMA,
            pltpu.SemaphoreType.DMA,
            pltpu.SemaphoreType.DMA,
            pltpu.SemaphoreType.DMA,
        ],
    )(_sc_body)
    return k(xt)


def kernel(features, labels, cams, intra_anchors, cross_anchors, epoch, lr):
    xt = jnp.transpose(intra_anchors, (1, 0, 2))
    cp, nm = _sc_normalize(xt)
    loss = jnp.asarray(epoch, jnp.float32) * 0.0
    return (
        loss,
        jnp.transpose(cp, (1, 0, 2)),
        jnp.transpose(nm, (1, 0, 2)),
    )


# speedup vs baseline: 18.7497x; 18.7497x over previous
import functools

import jax
import jax.numpy as jnp
from jax import lax
from jax.experimental import pallas as pl
from jax.experimental.pallas import tpu as pltpu
from jax.experimental.pallas import tpu_sc as plsc

_NUM_CAMS = 8
_NUM_IDS = 1500
_D = 2048

_NW = 32
_G = 2
_SLOTS = 3
_NCHUNKS = _NUM_IDS // _G
_STEPS = -(-_NCHUNKS // _NW)
_PAIRS = -(-_STEPS // _SLOTS)

_LANES = _D // 16


def _lanesum(acc):
    idx = lax.iota(jnp.int32, 16)
    for k in (1, 2, 4, 8):
        perm = acc.at[idx ^ k].get(mode="promise_in_bounds")
        acc = acc + perm
    return acc


def _rsqrt16(s):
    i = lax.bitcast_convert_type(s, jnp.int32)
    i = jnp.int32(0x5F3759DF) - lax.shift_right_arithmetic(i, 1)
    r = lax.bitcast_convert_type(i, jnp.float32)
    for _ in range(3):
        r = r * (1.5 - 0.5 * s * r * r)
    return r


def _sumsq_rows(buf):
    invs = []
    for g in range(_G):
        for cam in range(_NUM_CAMS):
            def sumsq(j, acc):
                v = buf[g, cam, pl.ds(j * 16, 16)]
                return acc + v * v

            acc = lax.fori_loop(0, _LANES, sumsq,
                                jnp.zeros((16,), jnp.float32), unroll=8)
            s = _lanesum(acc)
            invs.append(1.0 / (s * _rsqrt16(s) + 1e-12))
    return invs


def _scale_rows(buf, invs):
    for g in range(_G):
        for cam in range(_NUM_CAMS):
            inv = invs[g * _NUM_CAMS + cam]

            def scale(j, carry):
                v = buf[g, cam, pl.ds(j * 16, 16)]
                buf[g, cam, pl.ds(j * 16, 16)] = v * inv
                return carry

            lax.fori_loop(0, _LANES, scale, 0, unroll=8)


def _sc_body(x_hbm, cp_hbm, nm_hbm, *scratch):
    bufs = scratch[:_SLOTS]
    in_sems = scratch[_SLOTS:2 * _SLOTS]
    cp_sems = scratch[2 * _SLOTS:3 * _SLOTS]
    nm_sems = scratch[3 * _SLOTS:4 * _SLOTS]
    wid = lax.axis_index("s") * 2 + lax.axis_index("c")

    def src(cid):
        return x_hbm.at[pl.ds(cid * _G, _G)]

    for b in range(_SLOTS):
        cid = b * _NW + wid

        @pl.when(cid < _NCHUNKS)
        def _(b=b, cid=cid):
            pltpu.async_copy(src(cid), bufs[b], in_sems[b])

    def group(i, carry):
        for b in range(_SLOTS):
            t = i * _SLOTS + b
            cid = t * _NW + wid

            @pl.when(cid < _NCHUNKS)
            def _(b=b, cid=cid):
                buf = bufs[b]
                pltpu.make_async_copy(src(cid), buf, in_sems[b]).wait()
                cp_dma = pltpu.async_copy(buf, cp_hbm.at[pl.ds(cid * _G, _G)],
                                          cp_sems[b])
                invs = _sumsq_rows(buf)
                cp_dma.wait()
                _scale_rows(buf, invs)
                pltpu.async_copy(buf, nm_hbm.at[pl.ds(cid * _G, _G)],
                                 nm_sems[b])

            cid2 = cid + _SLOTS * _NW

            @pl.when(cid2 < _NCHUNKS)
            def _(b=b, cid=cid, cid2=cid2):
                pltpu.make_async_copy(bufs[b], nm_hbm.at[pl.ds(cid * _G, _G)],
                                      nm_sems[b]).wait()
                pltpu.async_copy(src(cid2), bufs[b], in_sems[b])

        return carry

    lax.fori_loop(0, _PAIRS, group, 0)

    for t in range(max(0, _STEPS - _SLOTS - 1), _STEPS):
        cid = t * _NW + wid

        @pl.when((cid < _NCHUNKS) & (cid + _SLOTS * _NW >= _NCHUNKS))
        def _(t=t, cid=cid):
            b = t % _SLOTS
            pltpu.make_async_copy(bufs[b], nm_hbm.at[pl.ds(cid * _G, _G)],
                                  nm_sems[b]).wait()


def _sc_normalize(xt):
    mesh = plsc.VectorSubcoreMesh(core_axis_name="c", subcore_axis_name="s")
    shape = jax.ShapeDtypeStruct((_NUM_IDS, _NUM_CAMS, _D), jnp.float32)
    k = functools.partial(
        pl.kernel,
        mesh=mesh,
        out_type=[shape, shape],
        scratch_types=(
            [pltpu.VMEM((_G, _NUM_CAMS, _D), jnp.float32)] * _SLOTS
            + [pltpu.SemaphoreType.DMA] * (3 * _SLOTS)
        ),
    )(_sc_body)
    return k(xt)


def kernel(features, labels, cams, intra_anchors, cross_anchors, epoch, lr):
    xt = jnp.transpose(intra_anchors, (1, 0, 2))
    cp, nm = _sc_normalize(xt)
    loss = jnp.asarray(epoch, jnp.float32) * 0.0
    return (
        loss,
        jnp.transpose(cp, (1, 0, 2)),
        jnp.transpose(nm, (1, 0, 2)),
    )
